# baseline (device time: 1066774 ns/iter reference)
import jax
import jax.numpy as jnp
from jax import lax
from jax.experimental import pallas as pl
from jax.experimental.pallas import tpu as pltpu

N_CHUNKS = 8


def kernel(x):
    m_per, n = x.shape
    n_half = n // 2
    m_tot = 2 * m_per
    m_half = m_per // 2
    rows_c = m_half // N_CHUNKS

    def body(x_ref, out_ref, ysend_sems, yrecv_sems, fsend_sems, frecv_sems,
             copy_sem):
        my_x = lax.axis_index("x")
        my_y = lax.axis_index("y")
        peer_y = 1 - my_y
        peer_x = 1 - my_x

        barrier_sem = pltpu.get_barrier_semaphore()
        pl.semaphore_signal(
            barrier_sem, inc=1, device_id=(my_x, peer_y),
            device_id_type=pl.DeviceIdType.MESH,
        )
        pl.semaphore_signal(
            barrier_sem, inc=1, device_id=(peer_x, my_y),
            device_id_type=pl.DeviceIdType.MESH,
        )
        pl.semaphore_wait(barrier_sem, 2)

        local = pltpu.make_async_copy(
            x_ref.at[:, pl.ds(my_y * n_half, n_half)],
            out_ref.at[pl.ds(my_y * m_per, m_per), :],
            copy_sem,
        )
        local.start()

        def y_rdma(c):
            return pltpu.make_async_remote_copy(
                src_ref=x_ref.at[
                    pl.ds(my_x * m_half + c * rows_c, rows_c),
                    pl.ds(peer_y * n_half, n_half),
                ],
                dst_ref=out_ref.at[
                    pl.ds(my_y * m_per + my_x * m_half + c * rows_c, rows_c), :
                ],
                send_sem=ysend_sems.at[c],
                recv_sem=yrecv_sems.at[c],
                device_id=(my_x, peer_y),
                device_id_type=pl.DeviceIdType.MESH,
            )

        def fwd_rdma(c):
            return pltpu.make_async_remote_copy(
                src_ref=out_ref.at[
                    pl.ds(peer_y * m_per + my_x * m_half + c * rows_c, rows_c), :
                ],
                dst_ref=out_ref.at[
                    pl.ds(peer_y * m_per + my_x * m_half + c * rows_c, rows_c), :
                ],
                send_sem=fsend_sems.at[c],
                recv_sem=frecv_sems.at[c],
                device_id=(peer_x, my_y),
                device_id_type=pl.DeviceIdType.MESH,
            )

        y_rdmas = [y_rdma(c) for c in range(N_CHUNKS)]
        fwd_rdmas = [fwd_rdma(c) for c in range(N_CHUNKS)]

        for r in y_rdmas:
            r.start()
        for c in range(N_CHUNKS):
            y_rdmas[c].wait_recv()
            fwd_rdmas[c].start()
        for r in y_rdmas:
            r.wait_send()
        for r in fwd_rdmas:
            r.wait_send()
        for r in fwd_rdmas:
            r.wait_recv()
        local.wait()

    return pl.pallas_call(
        body,
        out_shape=jax.ShapeDtypeStruct((m_tot, n_half), x.dtype),
        in_specs=[pl.BlockSpec(memory_space=pltpu.MemorySpace.HBM)],
        out_specs=pl.BlockSpec(memory_space=pltpu.MemorySpace.HBM),
        scratch_shapes=[
            pltpu.SemaphoreType.DMA((N_CHUNKS,)),
            pltpu.SemaphoreType.DMA((N_CHUNKS,)),
            pltpu.SemaphoreType.DMA((N_CHUNKS,)),
            pltpu.SemaphoreType.DMA((N_CHUNKS,)),
            pltpu.SemaphoreType.DMA,
        ],
        compiler_params=pltpu.CompilerParams(collective_id=0),
    )(x)


# device time: 261654 ns/iter; 4.0770x vs baseline; 4.0770x over previous
import jax
import jax.numpy as jnp
from jax import lax
from jax.experimental import pallas as pl
from jax.experimental.pallas import tpu as pltpu

N_CHUNKS = 8
N_LOCAL = 8


def kernel(x):
    m_per, n = x.shape
    n_half = n // 2
    m_tot = 2 * m_per
    m_half = m_per // 2
    rows_c = m_half // N_CHUNKS
    rows_l = m_per // N_LOCAL

    def body(x_ref, out_ref, ysend_sems, yrecv_sems, fsend_sems, frecv_sems,
             stage, in_sems, out_sems):
        my_x = lax.axis_index("x")
        my_y = lax.axis_index("y")
        peer_y = 1 - my_y
        peer_x = 1 - my_x

        barrier_sem = pltpu.get_barrier_semaphore()
        pl.semaphore_signal(
            barrier_sem, inc=1, device_id=(my_x, peer_y),
            device_id_type=pl.DeviceIdType.MESH,
        )
        pl.semaphore_signal(
            barrier_sem, inc=1, device_id=(peer_x, my_y),
            device_id_type=pl.DeviceIdType.MESH,
        )
        pl.semaphore_wait(barrier_sem, 2)

        def local_in(k):
            return pltpu.make_async_copy(
                x_ref.at[pl.ds(k * rows_l, rows_l), pl.ds(my_y * n_half, n_half)],
                stage.at[k % 2],
                in_sems.at[k % 2],
            )

        def local_out(k):
            return pltpu.make_async_copy(
                stage.at[k % 2],
                out_ref.at[pl.ds(my_y * m_per + k * rows_l, rows_l), :],
                out_sems.at[k % 2],
            )

        def y_rdma(c):
            return pltpu.make_async_remote_copy(
                src_ref=x_ref.at[
                    pl.ds(my_x * m_half + c * rows_c, rows_c),
                    pl.ds(peer_y * n_half, n_half),
                ],
                dst_ref=out_ref.at[
                    pl.ds(my_y * m_per + my_x * m_half + c * rows_c, rows_c), :
                ],
                send_sem=ysend_sems.at[c],
                recv_sem=yrecv_sems.at[c],
                device_id=(my_x, peer_y),
                device_id_type=pl.DeviceIdType.MESH,
            )

        def fwd_rdma(c):
            return pltpu.make_async_remote_copy(
                src_ref=out_ref.at[
                    pl.ds(peer_y * m_per + my_x * m_half + c * rows_c, rows_c), :
                ],
                dst_ref=out_ref.at[
                    pl.ds(peer_y * m_per + my_x * m_half + c * rows_c, rows_c), :
                ],
                send_sem=fsend_sems.at[c],
                recv_sem=frecv_sems.at[c],
                device_id=(peer_x, my_y),
                device_id_type=pl.DeviceIdType.MESH,
            )

        y_rdmas = [y_rdma(c) for c in range(N_CHUNKS)]
        fwd_rdmas = [fwd_rdma(c) for c in range(N_CHUNKS)]

        for r in y_rdmas:
            r.start()

        local_in(0).start()
        for k in range(N_LOCAL):
            local_in(k).wait()
            local_out(k).start()
            if k + 1 < N_LOCAL:
                if k >= 1:
                    local_out(k - 1).wait()
                local_in(k + 1).start()
        local_out(N_LOCAL - 2).wait()
        local_out(N_LOCAL - 1).wait()

        for c in range(N_CHUNKS):
            y_rdmas[c].wait_recv()
            fwd_rdmas[c].start()
        for r in y_rdmas:
            r.wait_send()
        for r in fwd_rdmas:
            r.wait_send()
        for r in fwd_rdmas:
            r.wait_recv()

    return pl.pallas_call(
        body,
        out_shape=jax.ShapeDtypeStruct((m_tot, n_half), x.dtype),
        in_specs=[pl.BlockSpec(memory_space=pltpu.MemorySpace.HBM)],
        out_specs=pl.BlockSpec(memory_space=pltpu.MemorySpace.HBM),
        scratch_shapes=[
            pltpu.SemaphoreType.DMA((N_CHUNKS,)),
            pltpu.SemaphoreType.DMA((N_CHUNKS,)),
            pltpu.SemaphoreType.DMA((N_CHUNKS,)),
            pltpu.SemaphoreType.DMA((N_CHUNKS,)),
            pltpu.VMEM((2, m_per // N_LOCAL, n // 2), x.dtype),
            pltpu.SemaphoreType.DMA((2,)),
            pltpu.SemaphoreType.DMA((2,)),
        ],
        compiler_params=pltpu.CompilerParams(collective_id=0),
    )(x)


# device time: 244267 ns/iter; 4.3672x vs baseline; 1.0712x over previous
import jax
import jax.numpy as jnp
from jax import lax
from jax.experimental import pallas as pl
from jax.experimental.pallas import tpu as pltpu

N_CHUNKS = 16
N_LOCAL = 8


def kernel(x):
    m_per, n = x.shape
    n_half = n // 2
    m_tot = 2 * m_per
    m_half = m_per // 2
    rows_c = m_half // N_CHUNKS
    rows_l = m_per // N_LOCAL

    def body(x_ref, out_ref, ysend_sems, yrecv_sems, fsend_sems, frecv_sems,
             stage, in_sems, out_sems):
        my_x = lax.axis_index("x")
        my_y = lax.axis_index("y")
        peer_y = 1 - my_y
        peer_x = 1 - my_x

        barrier_sem = pltpu.get_barrier_semaphore()
        pl.semaphore_signal(
            barrier_sem, inc=1, device_id=(my_x, peer_y),
            device_id_type=pl.DeviceIdType.MESH,
        )
        pl.semaphore_signal(
            barrier_sem, inc=1, device_id=(peer_x, my_y),
            device_id_type=pl.DeviceIdType.MESH,
        )
        pl.semaphore_wait(barrier_sem, 2)

        def local_in(k):
            return pltpu.make_async_copy(
                x_ref.at[pl.ds(k * rows_l, rows_l), pl.ds(my_y * n_half, n_half)],
                stage.at[k % 2],
                in_sems.at[k % 2],
            )

        def local_out(k):
            return pltpu.make_async_copy(
                stage.at[k % 2],
                out_ref.at[pl.ds(my_y * m_per + k * rows_l, rows_l), :],
                out_sems.at[k % 2],
            )

        def y_rdma(c):
            return pltpu.make_async_remote_copy(
                src_ref=x_ref.at[
                    pl.ds(my_x * m_half + c * rows_c, rows_c),
                    pl.ds(peer_y * n_half, n_half),
                ],
                dst_ref=out_ref.at[
                    pl.ds(my_y * m_per + my_x * m_half + c * rows_c, rows_c), :
                ],
                send_sem=ysend_sems.at[c],
                recv_sem=yrecv_sems.at[c],
                device_id=(my_x, peer_y),
                device_id_type=pl.DeviceIdType.MESH,
            )

        def fwd_rdma(c):
            return pltpu.make_async_remote_copy(
                src_ref=out_ref.at[
                    pl.ds(peer_y * m_per + my_x * m_half + c * rows_c, rows_c), :
                ],
                dst_ref=out_ref.at[
                    pl.ds(peer_y * m_per + my_x * m_half + c * rows_c, rows_c), :
                ],
                send_sem=fsend_sems.at[c],
                recv_sem=frecv_sems.at[c],
                device_id=(peer_x, my_y),
                device_id_type=pl.DeviceIdType.MESH,
            )

        y_rdmas = [y_rdma(c) for c in range(N_CHUNKS)]
        fwd_rdmas = [fwd_rdma(c) for c in range(N_CHUNKS)]

        for r in y_rdmas:
            r.start()

        local_in(0).start()
        for c in range(N_CHUNKS):
            y_rdmas[c].wait_recv()
            fwd_rdmas[c].start()
            if c < N_LOCAL:
                local_in(c).wait()
                local_out(c).start()
                if c + 1 < N_LOCAL:
                    if c >= 1:
                        local_out(c - 1).wait()
                    local_in(c + 1).start()
        local_out(N_LOCAL - 2).wait()
        local_out(N_LOCAL - 1).wait()
        for r in y_rdmas:
            r.wait_send()
        for r in fwd_rdmas:
            r.wait_send()
        for r in fwd_rdmas:
            r.wait_recv()

    return pl.pallas_call(
        body,
        out_shape=jax.ShapeDtypeStruct((m_tot, n_half), x.dtype),
        in_specs=[pl.BlockSpec(memory_space=pltpu.MemorySpace.HBM)],
        out_specs=pl.BlockSpec(memory_space=pltpu.MemorySpace.HBM),
        scratch_shapes=[
            pltpu.SemaphoreType.DMA((N_CHUNKS,)),
            pltpu.SemaphoreType.DMA((N_CHUNKS,)),
            pltpu.SemaphoreType.DMA((N_CHUNKS,)),
            pltpu.SemaphoreType.DMA((N_CHUNKS,)),
            pltpu.VMEM((2, m_per // N_LOCAL, n // 2), x.dtype),
            pltpu.SemaphoreType.DMA((2,)),
            pltpu.SemaphoreType.DMA((2,)),
        ],
        compiler_params=pltpu.CompilerParams(collective_id=0),
    )(x)
